# baseline (speedup 1.0000x reference)
import functools

import jax
import jax.numpy as jnp
from jax import lax
from jax.experimental import pallas as pl
from jax.experimental.pallas import tpu as pltpu
from jax.experimental.pallas import tpu_sc as plsc

N = 10000
E = 320000
D = 128
L = 5
POOL_AT = 3
K = 5000
NPAD = 10240
ROWS_PER_TILE = NPAD // 16
CHUNK = 128
NCHUNKS_E = E // CHUNK
NCHUNKS_EP = 2560
NCHUNKS_N = NPAD // CHUNK
INT_MIN32 = -(2 ** 31)

@functools.cache
def _mesh():
    return plsc.VectorSubcoreMesh(core_axis_name="c", subcore_axis_name="s")


def _atomenc(embf, xoff):
    CA = 128
    NCA = NPAD // CA

    @functools.partial(
        pl.kernel,
        out_type=jax.ShapeDtypeStruct((NPAD, D), jnp.float32),
        mesh=_mesh(),
        scratch_types=[
            pltpu.VMEM((9 * CA,), jnp.int32),
            pltpu.VMEM((2, CA, D), jnp.float32),
            pltpu.SemaphoreType.DMA,
            pltpu.SemaphoreType.DMA,
        ],
    )
    def body(embf_hbm, xoff_hbm, h0_hbm, idx_v, acc_v, gsem, wsem):
        c = lax.axis_index("c")
        s = lax.axis_index("s")
        w = s * 2 + c
        nchunk = jnp.where(w < NCA - 2 * 32, 3, 2)

        def step(j, carry):
            b = lax.rem(j, 2)
            chunk = w + j * 32
            pltpu.sync_copy(xoff_hbm.at[pl.ds(chunk * 9 * CA, 9 * CA)], idx_v)
            @pl.when(j >= 2)
            def _():
                pltpu.make_async_copy(
                    acc_v.at[0], h0_hbm.at[pl.ds(0, CA)], wsem).wait()

            pltpu.sync_copy(embf_hbm.at[idx_v.at[pl.ds(0, CA)]], acc_v.at[b])
            for i in range(1, 9):
                pltpu.async_copy(
                    embf_hbm.at[idx_v.at[pl.ds(i * CA, CA)]], acc_v.at[b],
                    gsem, add=True)
            for i in range(1, 9):
                pltpu.make_async_copy(
                    embf_hbm.at[idx_v.at[pl.ds(CA, CA)]], acc_v.at[b],
                    gsem).wait()
            pltpu.async_copy(acc_v.at[b], h0_hbm.at[pl.ds(chunk * CA, CA)], wsem)
            return carry

        lax.fori_loop(0, nchunk, step, 0)
        for _ in range(2):
            pltpu.make_async_copy(
                acc_v.at[0], h0_hbm.at[pl.ds(0, CA)], wsem).wait()

    return body(embf, xoff)


def _segsum(h, src, dst, zeros_slab):
    CPT = NCHUNKS_EP // 32
    PHASE = 40

    @functools.partial(
        pl.kernel,
        out_type=jax.ShapeDtypeStruct((2, NPAD, D), jnp.float32),
        mesh=_mesh(),
        scratch_types=[
            pltpu.VMEM((PHASE, CHUNK), jnp.int32),
            pltpu.VMEM((PHASE, CHUNK), jnp.int32),
            pltpu.VMEM((2, CHUNK, D), jnp.float32),
            pltpu.VMEM_SHARED((NPAD, D), jnp.float32),
            pltpu.SemaphoreType.DMA,
            pltpu.SemaphoreType.DMA,
        ],
    )
    def body(h_hbm, src_hbm, dst_hbm, z_hbm, out_hbm, sidx_v, didx_v, rows_v,
             agg_sp, gsem, ssem):
        c = lax.axis_index("c")
        s = lax.axis_index("s")
        w = c * 16 + s
        pltpu.sync_copy(z_hbm, agg_sp.at[pl.ds(s * ROWS_PER_TILE, ROWS_PER_TILE)])
        plsc.subcore_barrier()

        for p in range(CPT // PHASE):
            pltpu.sync_copy(src_hbm.at[pl.ds(w * CPT + p * PHASE, PHASE)], sidx_v)
            pltpu.sync_copy(dst_hbm.at[pl.ds(w * CPT + p * PHASE, PHASE)], didx_v)
            pltpu.async_copy(h_hbm.at[sidx_v.at[0]], rows_v.at[0], gsem)

            def step(j, carry):
                b = lax.rem(j, 2)

                @pl.when(j >= 1)
                def _():
                    pltpu.make_async_copy(
                        rows_v.at[0], agg_sp.at[didx_v.at[0]], ssem).wait()

                @pl.when(j + 1 < PHASE)
                def _():
                    pltpu.async_copy(
                        h_hbm.at[sidx_v.at[j + 1]], rows_v.at[1 - b], gsem)

                pltpu.make_async_copy(
                    h_hbm.at[sidx_v.at[j]], rows_v.at[b], gsem).wait()
                pltpu.async_copy(rows_v.at[b], agg_sp.at[didx_v.at[j]], ssem,
                                 add=True)
                return carry

            lax.fori_loop(0, PHASE, step, 0)
            pltpu.make_async_copy(
                rows_v.at[0], agg_sp.at[didx_v.at[0]], ssem).wait()
        plsc.subcore_barrier()
        pltpu.sync_copy(
            agg_sp.at[pl.ds(s * ROWS_PER_TILE, ROWS_PER_TILE)],
            out_hbm.at[c, pl.ds(s * ROWS_PER_TILE, ROWS_PER_TILE)],
        )

    return body(h, src, dst, zeros_slab)


def _enc_i32(x):
    b = lax.bitcast_convert_type(x, jnp.int32)
    return b ^ (lax.shift_right_arithmetic(b, 31) & jnp.int32(0x7FFFFFFF))


def _dense_body(h_ref, agg_ref, W_ref, b_ref, g_ref, be_ref, eps_ref, *rest,
                masked, mode):
    if masked:
        m_ref = rest[0]
        rest = rest[1:]
    h = h_ref[0:N] if h_ref.shape[0] != N else h_ref[...]
    agg = agg_ref[0, 0:N] + agg_ref[1, 0:N]
    z = eps_ref[0, 0] * h + agg
    z = jnp.dot(z, W_ref[...], preferred_element_type=jnp.float32) + b_ref[...]
    if masked:
        mm = m_ref[...]
        kdiv = float(K)
        mu = jnp.sum(z * mm, axis=0, keepdims=True) * (1.0 / kdiv)
        zc = z - mu
        var = jnp.sum(zc * zc * mm, axis=0, keepdims=True) * (1.0 / kdiv)
    else:
        kdiv = float(N)
        mu = jnp.sum(z, axis=0, keepdims=True) * (1.0 / kdiv)
        zc = z - mu
        var = jnp.sum(zc * zc, axis=0, keepdims=True) * (1.0 / kdiv)
    hp = jnp.maximum(zc / jnp.sqrt(var + 1e-5) * g_ref[...] + be_ref[...], 0.0)
    if masked:
        hp = hp * mm
    if mode == "final":
        outW_ref, outb_ref, o_ref = rest
        pooled = jnp.sum(hp, axis=0, keepdims=True) * (1.0 / kdiv)
        o = jnp.dot(pooled, outW_ref[...], preferred_element_type=jnp.float32)
        o_ref[...] = jax.nn.sigmoid(o + outb_ref[...])
    elif mode == "score":
        pw_ref, o_ref, sc_ref = rest
        o_ref[...] = hp
        pw = pw_ref[...]
        nrm = jnp.sqrt(jnp.sum(pw * pw))
        sc_ref[...] = jnp.tanh(
            jnp.dot(hp, pw, preferred_element_type=jnp.float32) / nrm)
    else:
        o_ref = rest[0]
        o_ref[...] = hp


def _dense(h, agg, W, b, g, be, eps1):
    return pl.pallas_call(
        functools.partial(_dense_body, masked=False, mode="plain"),
        out_shape=jax.ShapeDtypeStruct((N, D), jnp.float32),
    )(h, agg, W, b, g, be, eps1)


def _dense_score(h, agg, W, b, g, be, eps1, pw):
    return pl.pallas_call(
        functools.partial(_dense_body, masked=False, mode="score"),
        out_shape=(jax.ShapeDtypeStruct((N, D), jnp.float32),
                   jax.ShapeDtypeStruct((N, 1), jnp.float32)),
    )(h, agg, W, b, g, be, eps1, pw)


def _dense_masked(h, agg, W, b, g, be, eps1, m):
    return pl.pallas_call(
        functools.partial(_dense_body, masked=True, mode="plain"),
        out_shape=jax.ShapeDtypeStruct((N, D), jnp.float32),
    )(h, agg, W, b, g, be, eps1, m)


def _dense_final(h, agg, W, b, g, be, eps1, m, outW, outb):
    return pl.pallas_call(
        functools.partial(_dense_body, masked=True, mode="final"),
        out_shape=jax.ShapeDtypeStruct((1, 1), jnp.float32),
    )(h, agg, W, b, g, be, eps1, m, outW, outb)


def _pool_body(h_ref, s2d_ref, scol_ref, hm_ref, m_ref):
    e2d = _enc_i32(s2d_ref[...])
    t = jnp.full((1, 1), INT_MIN32, jnp.int32)
    for bit in range(31, -1, -1):
        cand = t + jnp.int32(INT_MIN32 if bit == 31 else 1 << bit)
        cnt = jnp.sum((e2d >= cand).astype(jnp.int32))
        t = jnp.where(cnt >= K, cand, t)
    need = K - jnp.sum((e2d > t).astype(jnp.int32))
    eq2d = e2d == t
    r0 = lax.broadcasted_iota(jnp.int32, (NPAD // 128, 128), 0)
    r1 = lax.broadcasted_iota(jnp.int32, (NPAD // 128, 128), 1)
    idx2d = r0 * 128 + r1
    c0 = jnp.zeros((1, 1), jnp.int32)
    for bit in range(13, -1, -1):
        cand = c0 + jnp.int32(1 << bit)
        cnt = jnp.sum((eq2d & (idx2d < cand)).astype(jnp.int32))
        c0 = jnp.where(cnt < need, cand, c0)
    scol = scol_ref[...]
    ecol = _enc_i32(scol)
    idxc = lax.broadcasted_iota(jnp.int32, (N, 1), 0)
    sel = (ecol > t) | ((ecol == t) & (idxc <= c0))
    mf = sel.astype(jnp.float32)
    m_ref[...] = mf
    hm_ref[...] = h_ref[...] * (scol * mf)


def _pool(h, s2d, scol):
    return pl.pallas_call(
        _pool_body,
        out_shape=(jax.ShapeDtypeStruct((N, D), jnp.float32),
                   jax.ShapeDtypeStruct((N, 1), jnp.float32)),
    )(h, s2d, scol)


def kernel(atom_emb, bond_emb, conv_W, conv_b, bn_gamma, bn_beta, eps, pool_w,
           out_W, out_b, x, edge_index, edge_attr, batch):
    x32 = x.astype(jnp.int32)
    xoff = x32 + jnp.arange(9, dtype=jnp.int32)[None, :] * atom_emb.shape[1]
    xoff = jnp.concatenate(
        [xoff, jnp.zeros((NPAD - N, 9), jnp.int32)], axis=0)
    xoff = xoff.reshape(NPAD // 128, 128, 9).transpose(0, 2, 1)
    xoff = xoff + (jnp.arange(NPAD // 128, dtype=jnp.int32) % 8 * (9 * 128)
                   )[:, None, None]
    xoff = xoff.reshape(-1)
    embf = atom_emb.reshape(9 * atom_emb.shape[1], D)
    embf = jnp.concatenate([embf] * 8)
    npadc = NCHUNKS_EP - NCHUNKS_E
    pad_src = jnp.arange(npadc * CHUNK, dtype=jnp.int32) * 7 % N
    src = jnp.concatenate(
        [edge_index[0].astype(jnp.int32).reshape(NCHUNKS_E, CHUNK),
         pad_src.reshape(npadc, CHUNK)])
    pad_dst = N + jnp.arange(npadc * CHUNK, dtype=jnp.int32) % (NPAD - N)
    dst = jnp.concatenate(
        [edge_index[1].astype(jnp.int32).reshape(NCHUNKS_E, CHUNK),
         pad_dst.reshape(npadc, CHUNK)])
    zeros_slab = jnp.zeros((ROWS_PER_TILE, D), jnp.float32)
    eps1 = (1.0 + eps).reshape(L, 1, 1)
    cb = conv_b.reshape(L, 1, D)
    cg = bn_gamma.reshape(L, 1, D)
    cbe = bn_beta.reshape(L, 1, D)
    pw = pool_w.reshape(D, 1)

    h = _atomenc(embf, xoff)
    m = None
    for layer in range(L):
        if layer == POOL_AT:
            spad = jnp.concatenate(
                [scol, jnp.full((NPAD - N, 1), -1e30, jnp.float32)], axis=0)
            s2d = spad.reshape(NPAD // 128, 128)
            h, m = _pool(h, s2d, scol)
        agg = _segsum(h, src, dst, zeros_slab)
        args = (h, agg, conv_W[layer], cb[layer], cg[layer], cbe[layer], eps1[layer])
        if layer == POOL_AT - 1:
            h, scol = _dense_score(*args, pw)
        elif layer == L - 1:
            return _dense_final(*args, m, out_W, out_b.reshape(1, 1))
        elif layer > POOL_AT - 1:
            h = _dense_masked(*args, m)
        else:
            h = _dense(*args)


# speedup vs baseline: 1.0188x; 1.0188x over previous
import functools

import jax
import jax.numpy as jnp
from jax import lax
from jax.experimental import pallas as pl
from jax.experimental.pallas import tpu as pltpu
from jax.experimental.pallas import tpu_sc as plsc

N = 10000
E = 320000
D = 128
L = 5
POOL_AT = 3
K = 5000
NPAD = 10240
ROWS_PER_TILE = NPAD // 16
CHUNK = 128
NCHUNKS_E = E // CHUNK
NCHUNKS_EP = 2560
NCHUNKS_N = NPAD // CHUNK
INT_MIN32 = -(2 ** 31)

@functools.cache
def _mesh():
    return plsc.VectorSubcoreMesh(core_axis_name="c", subcore_axis_name="s")


def _atomenc(embf, xoff):
    CA = 64
    NCA = NPAD // CA

    @functools.partial(
        pl.kernel,
        out_type=jax.ShapeDtypeStruct((NPAD, D), jnp.float32),
        mesh=_mesh(),
        scratch_types=[
            pltpu.VMEM((9 * CA,), jnp.int32),
            pltpu.VMEM((9, CA, D), jnp.float32),
            pltpu.VMEM((2, CA, D), jnp.float32),
            pltpu.SemaphoreType.DMA,
            pltpu.SemaphoreType.DMA,
        ],
    )
    def body(embf_hbm, xoff_hbm, h0_hbm, idx_v, bufs_v, acc_v, gsem, wsem):
        c = lax.axis_index("c")
        s = lax.axis_index("s")
        w = s * 2 + c

        def step(j, carry):
            b = lax.rem(j, 2)
            chunk = w + j * 32
            pltpu.sync_copy(xoff_hbm.at[pl.ds(chunk * 9 * CA, 9 * CA)], idx_v)
            for i in range(9):
                pltpu.async_copy(
                    embf_hbm.at[idx_v.at[pl.ds(i * CA, CA)]], bufs_v.at[i], gsem)
            for i in range(9):
                pltpu.make_async_copy(
                    embf_hbm.at[idx_v.at[pl.ds(0, CA)]], bufs_v.at[0], gsem).wait()

            @pl.when(j >= 2)
            def _():
                pltpu.make_async_copy(
                    acc_v.at[0], h0_hbm.at[pl.ds(0, CA)], wsem).wait()

            def accum(r, carry2):
                for g in range(D // 16):
                    sl = pl.ds(g * 16, 16)
                    v = bufs_v[0, r, sl]
                    for i in range(1, 9):
                        v = v + bufs_v[i, r, sl]
                    acc_v[b, r, sl] = v
                return carry2

            lax.fori_loop(0, CA, accum, 0)
            pltpu.async_copy(acc_v.at[b], h0_hbm.at[pl.ds(chunk * CA, CA)], wsem)
            return carry

        lax.fori_loop(0, NCA // 32, step, 0)
        for _ in range(2):
            pltpu.make_async_copy(
                acc_v.at[0], h0_hbm.at[pl.ds(0, CA)], wsem).wait()

    return body(embf, xoff)


def _segsum(h, src, dst, zeros_slab):
    CPT = NCHUNKS_EP // 32
    PHASE = 40

    @functools.partial(
        pl.kernel,
        out_type=jax.ShapeDtypeStruct((2, NPAD, D), jnp.float32),
        mesh=_mesh(),
        scratch_types=[
            pltpu.VMEM((PHASE, CHUNK), jnp.int32),
            pltpu.VMEM((PHASE, CHUNK), jnp.int32),
            pltpu.VMEM((2, CHUNK, D), jnp.float32),
            pltpu.VMEM_SHARED((NPAD, D), jnp.float32),
            pltpu.SemaphoreType.DMA,
            pltpu.SemaphoreType.DMA,
        ],
    )
    def body(h_hbm, src_hbm, dst_hbm, z_hbm, out_hbm, sidx_v, didx_v, rows_v,
             agg_sp, gsem, ssem):
        c = lax.axis_index("c")
        s = lax.axis_index("s")
        w = c * 16 + s
        pltpu.sync_copy(z_hbm, agg_sp.at[pl.ds(s * ROWS_PER_TILE, ROWS_PER_TILE)])
        plsc.subcore_barrier()

        for p in range(CPT // PHASE):
            pltpu.sync_copy(src_hbm.at[pl.ds(w * CPT + p * PHASE, PHASE)], sidx_v)
            pltpu.sync_copy(dst_hbm.at[pl.ds(w * CPT + p * PHASE, PHASE)], didx_v)
            pltpu.async_copy(h_hbm.at[sidx_v.at[0]], rows_v.at[0], gsem)

            def step(j, carry):
                b = lax.rem(j, 2)

                @pl.when(j >= 1)
                def _():
                    pltpu.make_async_copy(
                        rows_v.at[0], agg_sp.at[didx_v.at[0]], ssem).wait()

                @pl.when(j + 1 < PHASE)
                def _():
                    pltpu.async_copy(
                        h_hbm.at[sidx_v.at[j + 1]], rows_v.at[1 - b], gsem)

                pltpu.make_async_copy(
                    h_hbm.at[sidx_v.at[j]], rows_v.at[b], gsem).wait()
                pltpu.async_copy(rows_v.at[b], agg_sp.at[didx_v.at[j]], ssem,
                                 add=True)
                return carry

            lax.fori_loop(0, PHASE, step, 0)
            pltpu.make_async_copy(
                rows_v.at[0], agg_sp.at[didx_v.at[0]], ssem).wait()
        plsc.subcore_barrier()
        pltpu.sync_copy(
            agg_sp.at[pl.ds(s * ROWS_PER_TILE, ROWS_PER_TILE)],
            out_hbm.at[c, pl.ds(s * ROWS_PER_TILE, ROWS_PER_TILE)],
        )

    return body(h, src, dst, zeros_slab)


def _enc_i32(x):
    b = lax.bitcast_convert_type(x, jnp.int32)
    return b ^ (lax.shift_right_arithmetic(b, 31) & jnp.int32(0x7FFFFFFF))


def _dense_body(h_ref, agg_ref, W_ref, b_ref, g_ref, be_ref, eps_ref, *rest,
                masked, mode):
    if masked:
        m_ref = rest[0]
        rest = rest[1:]
    h = h_ref[0:N] if h_ref.shape[0] != N else h_ref[...]
    agg = agg_ref[0, 0:N] + agg_ref[1, 0:N]
    z = eps_ref[0, 0] * h + agg
    z = jnp.dot(z, W_ref[...], preferred_element_type=jnp.float32) + b_ref[...]
    if masked:
        mm = m_ref[...]
        kdiv = float(K)
        mu = jnp.sum(z * mm, axis=0, keepdims=True) * (1.0 / kdiv)
        zc = z - mu
        var = jnp.sum(zc * zc * mm, axis=0, keepdims=True) * (1.0 / kdiv)
    else:
        kdiv = float(N)
        mu = jnp.sum(z, axis=0, keepdims=True) * (1.0 / kdiv)
        zc = z - mu
        var = jnp.sum(zc * zc, axis=0, keepdims=True) * (1.0 / kdiv)
    hp = jnp.maximum(zc / jnp.sqrt(var + 1e-5) * g_ref[...] + be_ref[...], 0.0)
    if masked:
        hp = hp * mm
    if mode == "final":
        outW_ref, outb_ref, o_ref = rest
        pooled = jnp.sum(hp, axis=0, keepdims=True) * (1.0 / kdiv)
        o = jnp.dot(pooled, outW_ref[...], preferred_element_type=jnp.float32)
        o_ref[...] = jax.nn.sigmoid(o + outb_ref[...])
    elif mode == "score":
        pw_ref, o_ref, sc_ref = rest
        o_ref[...] = hp
        pw = pw_ref[...]
        nrm = jnp.sqrt(jnp.sum(pw * pw))
        sc_ref[...] = jnp.tanh(
            jnp.dot(hp, pw, preferred_element_type=jnp.float32) / nrm)
    else:
        o_ref = rest[0]
        o_ref[...] = hp


def _dense(h, agg, W, b, g, be, eps1):
    return pl.pallas_call(
        functools.partial(_dense_body, masked=False, mode="plain"),
        out_shape=jax.ShapeDtypeStruct((N, D), jnp.float32),
    )(h, agg, W, b, g, be, eps1)


def _dense_score(h, agg, W, b, g, be, eps1, pw):
    return pl.pallas_call(
        functools.partial(_dense_body, masked=False, mode="score"),
        out_shape=(jax.ShapeDtypeStruct((N, D), jnp.float32),
                   jax.ShapeDtypeStruct((N, 1), jnp.float32)),
    )(h, agg, W, b, g, be, eps1, pw)


def _dense_masked(h, agg, W, b, g, be, eps1, m):
    return pl.pallas_call(
        functools.partial(_dense_body, masked=True, mode="plain"),
        out_shape=jax.ShapeDtypeStruct((N, D), jnp.float32),
    )(h, agg, W, b, g, be, eps1, m)


def _dense_final(h, agg, W, b, g, be, eps1, m, outW, outb):
    return pl.pallas_call(
        functools.partial(_dense_body, masked=True, mode="final"),
        out_shape=jax.ShapeDtypeStruct((1, 1), jnp.float32),
    )(h, agg, W, b, g, be, eps1, m, outW, outb)


def _pool_body(h_ref, s2d_ref, scol_ref, hm_ref, m_ref):
    e2d = _enc_i32(s2d_ref[...])
    t = jnp.full((1, 1), INT_MIN32, jnp.int32)
    for bit in range(31, -1, -1):
        cand = t + jnp.int32(INT_MIN32 if bit == 31 else 1 << bit)
        cnt = jnp.sum((e2d >= cand).astype(jnp.int32))
        t = jnp.where(cnt >= K, cand, t)
    need = K - jnp.sum((e2d > t).astype(jnp.int32))
    eq2d = e2d == t
    r0 = lax.broadcasted_iota(jnp.int32, (NPAD // 128, 128), 0)
    r1 = lax.broadcasted_iota(jnp.int32, (NPAD // 128, 128), 1)
    idx2d = r0 * 128 + r1
    c0 = jnp.zeros((1, 1), jnp.int32)
    for bit in range(13, -1, -1):
        cand = c0 + jnp.int32(1 << bit)
        cnt = jnp.sum((eq2d & (idx2d < cand)).astype(jnp.int32))
        c0 = jnp.where(cnt < need, cand, c0)
    scol = scol_ref[...]
    ecol = _enc_i32(scol)
    idxc = lax.broadcasted_iota(jnp.int32, (N, 1), 0)
    sel = (ecol > t) | ((ecol == t) & (idxc <= c0))
    mf = sel.astype(jnp.float32)
    m_ref[...] = mf
    hm_ref[...] = h_ref[...] * (scol * mf)


def _pool(h, s2d, scol):
    return pl.pallas_call(
        _pool_body,
        out_shape=(jax.ShapeDtypeStruct((N, D), jnp.float32),
                   jax.ShapeDtypeStruct((N, 1), jnp.float32)),
    )(h, s2d, scol)


def kernel(atom_emb, bond_emb, conv_W, conv_b, bn_gamma, bn_beta, eps, pool_w,
           out_W, out_b, x, edge_index, edge_attr, batch):
    x32 = x.astype(jnp.int32)
    xoff = x32 + jnp.arange(9, dtype=jnp.int32)[None, :] * atom_emb.shape[1]
    xoff = jnp.concatenate(
        [xoff, jnp.zeros((NPAD - N, 9), jnp.int32)], axis=0)
    xoff = xoff.reshape(NPAD // 64, 64, 9).transpose(0, 2, 1)
    xoff = xoff + (jnp.arange(NPAD // 64, dtype=jnp.int32) % 8 * (9 * 128)
                   )[:, None, None]
    xoff = xoff.reshape(-1)
    embf = atom_emb.reshape(9 * atom_emb.shape[1], D)
    embf = jnp.concatenate([embf] * 8)
    npadc = NCHUNKS_EP - NCHUNKS_E
    pad_src = jnp.arange(npadc * CHUNK, dtype=jnp.int32) * 7 % N
    src = jnp.concatenate(
        [edge_index[0].astype(jnp.int32).reshape(NCHUNKS_E, CHUNK),
         pad_src.reshape(npadc, CHUNK)])
    pad_dst = N + jnp.arange(npadc * CHUNK, dtype=jnp.int32) % (NPAD - N)
    dst = jnp.concatenate(
        [edge_index[1].astype(jnp.int32).reshape(NCHUNKS_E, CHUNK),
         pad_dst.reshape(npadc, CHUNK)])
    zeros_slab = jnp.zeros((ROWS_PER_TILE, D), jnp.float32)
    eps1 = (1.0 + eps).reshape(L, 1, 1)
    cb = conv_b.reshape(L, 1, D)
    cg = bn_gamma.reshape(L, 1, D)
    cbe = bn_beta.reshape(L, 1, D)
    pw = pool_w.reshape(D, 1)

    h = _atomenc(embf, xoff)
    m = None
    for layer in range(L):
        if layer == POOL_AT:
            spad = jnp.concatenate(
                [scol, jnp.full((NPAD - N, 1), -1e30, jnp.float32)], axis=0)
            s2d = spad.reshape(NPAD // 128, 128)
            h, m = _pool(h, s2d, scol)
        agg = _segsum(h, src, dst, zeros_slab)
        args = (h, agg, conv_W[layer], cb[layer], cg[layer], cbe[layer], eps1[layer])
        if layer == POOL_AT - 1:
            h, scol = _dense_score(*args, pw)
        elif layer == L - 1:
            return _dense_final(*args, m, out_W, out_b.reshape(1, 1))
        elif layer > POOL_AT - 1:
            h = _dense_masked(*args, m)
        else:
            h = _dense(*args)
